# baseline (speedup 1.0000x reference)
import jax
import jax.numpy as jnp
from jax.experimental import pallas as pl
from jax.experimental.pallas import tpu as pltpu
from jax.experimental.pallas import tpu_sc as plsc

_PACK_BLK = 16384
_K_BLOCKS = 30
_K = _K_BLOCKS * _PACK_BLK
_NP_BLOCKS = _K_BLOCKS + 2
_NP = _NP_BLOCKS * _PACK_BLK
_WINDOW = 512


def _pack_table(table_t):
    d, v = table_t.shape

    def body(ta_ref, tb_ref, tout_ref):
        tout_ref[...] = jnp.concatenate(
            [ta_ref[...], tb_ref[...]], axis=0
        ).T

    return pl.pallas_call(
        body,
        grid=(_NP_BLOCKS,),
        in_specs=[
            pl.BlockSpec((d, _PACK_BLK), lambda i: (0, i)),
            pl.BlockSpec((d, _PACK_BLK), lambda i: (0, i + _K_BLOCKS)),
        ],
        out_specs=pl.BlockSpec((_PACK_BLK, 2 * d), lambda i: (i, 0)),
        out_shape=jax.ShapeDtypeStruct((_NP, 2 * d), table_t.dtype),
        compiler_params=pltpu.CompilerParams(
            dimension_semantics=("parallel",)
        ),
    )(table_t, table_t)


def _gather_rows(table_rows, idx_flat):
    n_idx = idx_flat.shape[0]
    d = table_rows.shape[1]
    mesh = plsc.VectorSubcoreMesh(core_axis_name="c", subcore_axis_name="s")

    @pl.kernel(
        out_type=jax.ShapeDtypeStruct((n_idx, d), table_rows.dtype),
        mesh=mesh,
        compiler_params=pltpu.CompilerParams(use_tc_tiling_on_sc=False),
    )
    def sc_gather(table_hbm, idx_hbm, out_hbm):
        def body(idx_vmem, out_vmem):
            pltpu.sync_copy(table_hbm.at[idx_vmem], out_vmem)

        pltpu.emit_pipeline(
            body,
            grid=(n_idx // _WINDOW,),
            in_specs=[pl.BlockSpec((_WINDOW,), lambda i: (i,))],
            out_specs=[pl.BlockSpec((_WINDOW, d), lambda i: (i, 0))],
            core_axis_name=("c", "s"),
            dimension_semantics=(pltpu.PARALLEL,),
        )(idx_hbm, out_hbm)

    return sc_gather(table_rows, idx_flat)


def kernel(x, table):
    b, s = x.shape
    v, d = table.shape
    packed = _pack_table(table.T)
    idx = x.reshape(-1).astype(jnp.int32)
    idx2 = jnp.where(idx >= _K, 2 * (idx - _K) + 1, 2 * idx)
    out = _gather_rows(packed.reshape(2 * _NP, d), idx2)
    return out.reshape(b, s, d)
